# baseline (speedup 1.0000x reference)
import jax
import jax.numpy as jnp
from jax import lax
from jax.experimental import pallas as pl
from jax.experimental.pallas import tpu as pltpu
from jax.experimental.pallas import tpu_sc as plsc

_B, _F, _C = 1024, 26, 1000
_N = _F * _C
_NW = 32
_RCH = 40
_NCHT = _N // _RCH
_PAIRS = _NCHT // (2 * _NW)
_TAILW = _NCHT - 2 * _PAIRS * _NW


def _sc_body(cidxT_hbm, out_hbm, cidx, bufs, sems):
    wid = lax.axis_index("s") * 2 + lax.axis_index("c")
    ones = jnp.ones((16,), jnp.int32)
    zeros = jnp.zeros((16,), jnp.int32)
    i16 = lax.iota(jnp.int32, 16)

    def _zero(s):
        def _zb(r, carry):
            for g in range(_B // 16):
                bufs[s][r, pl.ds(g * 16, 16)] = zeros
            return carry
        lax.fori_loop(0, _RCH, _zb, None)

    def _scatter(s, j0, val):
        def _g(gg, carry):
            for u in range(4):
                g = gg * 4 + u
                bvec = g * 16 + i16
                cvec = cidx[s][pl.ds(pl.multiple_of(g * 16, 16), 16)]
                m = (cvec >= j0) & (cvec < j0 + _RCH)
                plsc.store_scatter(bufs[s], [cvec - j0, bvec], val, mask=m)
            return carry
        lax.fori_loop(0, _B // 64, _g, None)

    def _wait(s, j0p):
        pltpu.make_async_copy(
            bufs[s], out_hbm.at[pl.ds(j0p, _RCH)], sems[s]).wait()

    def _chunk(s, k):
        ci = wid + k * _NW
        j0 = ci * _RCH

        @pl.when(k > 1)
        def _drain():
            j0p = j0 - 2 * _NW * _RCH
            _wait(s, j0p)
            _scatter(s, j0p, zeros)

        pltpu.sync_copy(cidxT_hbm.at[j0 // _C], cidx[s])
        _scatter(s, j0, ones)
        pltpu.async_copy(bufs[s], out_hbm.at[pl.ds(j0, _RCH)], sems[s])

    _zero(0)
    _chunk(0, 0)
    _zero(1)

    def _pair(t, carry):
        _chunk(1, 2 * t + 1)
        _chunk(0, 2 * t + 2)
        return carry

    lax.fori_loop(0, _PAIRS - 1, _pair, None)
    _chunk(1, 2 * _PAIRS - 1)

    @pl.when(wid < _TAILW)
    def _tail():
        _chunk(0, 2 * _PAIRS)

    _wait(0, (wid + jnp.where(wid < _TAILW, 2 * _PAIRS, 2 * _PAIRS - 2)
              * _NW) * _RCH)
    _wait(1, (wid + (2 * _PAIRS - 1) * _NW) * _RCH)


def kernel(x):
    colidxT = (x + jnp.arange(_F, dtype=jnp.int32) * _C).T
    fn = pl.kernel(
        _sc_body,
        out_type=jax.ShapeDtypeStruct((_N, _B), jnp.int32),
        mesh=plsc.VectorSubcoreMesh(core_axis_name="c", subcore_axis_name="s"),
        scratch_types=[
            (pltpu.VMEM((_B,), jnp.int32), pltpu.VMEM((_B,), jnp.int32)),
            (pltpu.VMEM((_RCH, _B), jnp.int32),
             pltpu.VMEM((_RCH, _B), jnp.int32)),
            (pltpu.SemaphoreType.DMA, pltpu.SemaphoreType.DMA),
        ],
        compiler_params=pltpu.CompilerParams(needs_layout_passes=False),
    )
    return fn(colidxT).T
